# Initial kernel scaffold; baseline (speedup 1.0000x reference)
import jax
import jax.numpy as jnp
from jax.experimental import pallas as pl


def kernel(x, edge_index, mask, enc_mask_token, W_enc0, W_enc1, W_e2d, W_dec):
    raise NotImplementedError("write your pallas kernel here")


# speedup vs baseline: 1.3340x; 1.3340x over previous
import functools

import jax
import jax.numpy as jnp
from jax.experimental import pallas as pl
from jax.experimental.pallas import tpu as pltpu

_N = 10000
_D = 512
_BR = 1000
_EPS = 1e-8


def _prep_body(x_ref, m_ref, tok_ref, ds_ref, dd_ref,
               t0_ref, ns_ref, nd_ref, s2_ref):
    ns = jax.lax.rsqrt(jnp.maximum(ds_ref[...], 1.0))
    nd = jax.lax.rsqrt(jnp.maximum(dd_ref[...], 1.0))
    m = m_ref[...]
    xm = jnp.where(m > 0.5, tok_ref[...], x_ref[...])
    t0_ref[...] = xm * ns
    ns_ref[...] = ns
    nd_ref[...] = nd
    s2_ref[...] = jnp.where(m > 0.5, 0.0, ns)


def _prep(x, mf, tok, ds, dd):
    grid = _N // _BR
    row = lambda i: (i, 0)
    return pl.pallas_call(
        _prep_body,
        grid=(grid,),
        in_specs=[
            pl.BlockSpec((_BR, _D), row),
            pl.BlockSpec((_BR, 1), row),
            pl.BlockSpec((1, _D), lambda i: (0, 0)),
            pl.BlockSpec((_BR, 1), row),
            pl.BlockSpec((_BR, 1), row),
        ],
        out_specs=[
            pl.BlockSpec((_BR, _D), row),
            pl.BlockSpec((_BR, 1), row),
            pl.BlockSpec((_BR, 1), row),
            pl.BlockSpec((_BR, 1), row),
        ],
        out_shape=[
            jax.ShapeDtypeStruct((_N, _D), jnp.float32),
            jax.ShapeDtypeStruct((_N, 1), jnp.float32),
            jax.ShapeDtypeStruct((_N, 1), jnp.float32),
            jax.ShapeDtypeStruct((_N, 1), jnp.float32),
        ],
    )(x, mf, tok, ds, dd)


def _mm1_body(a_ref, nd_ref, ns_ref, w_ref, o_ref):
    h = jnp.dot(a_ref[...] * nd_ref[...], w_ref[...],
                preferred_element_type=jnp.float32)
    o_ref[...] = jax.nn.relu(h) * ns_ref[...]


def _mm1(a, nd, ns, w):
    grid = _N // _BR
    row = lambda i: (i, 0)
    return pl.pallas_call(
        _mm1_body,
        grid=(grid,),
        in_specs=[
            pl.BlockSpec((_BR, _D), row),
            pl.BlockSpec((_BR, 1), row),
            pl.BlockSpec((_BR, 1), row),
            pl.BlockSpec((_D, _D), lambda i: (0, 0)),
        ],
        out_specs=pl.BlockSpec((_BR, _D), row),
        out_shape=jax.ShapeDtypeStruct((_N, _D), jnp.float32),
    )(a, nd, ns, w)


def _mm2_body(a_ref, nd_ref, s2_ref, w1_ref, w2_ref, o_ref):
    h = jnp.dot(a_ref[...] * nd_ref[...], w1_ref[...],
                preferred_element_type=jnp.float32)
    h = jax.nn.relu(h)
    rep = jnp.dot(h, w2_ref[...], preferred_element_type=jnp.float32)
    o_ref[...] = rep * s2_ref[...]


def _mm2(a, nd, s2, w1, w2):
    grid = _N // _BR
    row = lambda i: (i, 0)
    return pl.pallas_call(
        _mm2_body,
        grid=(grid,),
        in_specs=[
            pl.BlockSpec((_BR, _D), row),
            pl.BlockSpec((_BR, 1), row),
            pl.BlockSpec((_BR, 1), row),
            pl.BlockSpec((_D, _D), lambda i: (0, 0)),
            pl.BlockSpec((_D, _D), lambda i: (0, 0)),
        ],
        out_specs=pl.BlockSpec((_BR, _D), row),
        out_shape=jax.ShapeDtypeStruct((_N, _D), jnp.float32),
    )(a, nd, s2, w1, w2)


def _final_body(a_ref, nd_ref, w_ref, x_ref, m_ref, o_ref, acc_ref):
    i = pl.program_id(0)

    @pl.when(i == 0)
    def _():
        acc_ref[0] = 0.0
        acc_ref[1] = 0.0

    r = jnp.dot(a_ref[...] * nd_ref[...], w_ref[...],
                preferred_element_type=jnp.float32)
    x = x_ref[...]
    xr = jnp.sum(x * r, axis=1, keepdims=True)
    xx = jnp.sum(x * x, axis=1, keepdims=True)
    rr = jnp.sum(r * r, axis=1, keepdims=True)
    cos = xr / ((jnp.sqrt(xx) + _EPS) * (jnp.sqrt(rr) + _EPS))
    per = (1.0 - cos) ** 2
    m = m_ref[...]
    acc_ref[0] += jnp.sum(per * m)
    acc_ref[1] += jnp.sum(m)

    @pl.when(i == pl.num_programs(0) - 1)
    def _():
        o_ref[...] = jnp.reshape(acc_ref[0] / jnp.maximum(acc_ref[1], 1.0),
                                 (1, 1))


def _final(a, nd, w, x, mf):
    grid = _N // _BR
    row = lambda i: (i, 0)
    return pl.pallas_call(
        _final_body,
        grid=(grid,),
        in_specs=[
            pl.BlockSpec((_BR, _D), row),
            pl.BlockSpec((_BR, 1), row),
            pl.BlockSpec((_D, _D), lambda i: (0, 0)),
            pl.BlockSpec((_BR, _D), row),
            pl.BlockSpec((_BR, 1), row),
        ],
        out_specs=pl.BlockSpec((1, 1), lambda i: (0, 0)),
        out_shape=jax.ShapeDtypeStruct((1, 1), jnp.float32),
        scratch_shapes=[pltpu.SMEM((2,), jnp.float32)],
    )(a, nd, w, x, mf)


def _propagate(h, src, dst):
    return jnp.zeros((_N, _D), jnp.float32).at[dst].add(h[src])


def kernel(x, edge_index, mask, enc_mask_token, W_enc0, W_enc1, W_e2d, W_dec):
    src = edge_index[0]
    dst = edge_index[1]
    mf = mask.astype(jnp.float32)[:, None]
    ds = jnp.bincount(src, length=_N).astype(jnp.float32)[:, None]
    dd = jnp.bincount(dst, length=_N).astype(jnp.float32)[:, None]
    t0, ns, nd, s2 = _prep(x, mf, enc_mask_token, ds, dd)
    a1 = _propagate(t0, src, dst)
    t1 = _mm1(a1, nd, ns, W_enc0)
    a2 = _propagate(t1, src, dst)
    t2 = _mm2(a2, nd, s2, W_enc1, W_e2d)
    a3 = _propagate(t2, src, dst)
    loss = _final(a3, nd, W_dec, x, mf)
    return loss.reshape(())


# speedup vs baseline: 9.1883x; 6.8877x over previous
import functools

import jax
import jax.numpy as jnp
from jax import lax
from jax.experimental import pallas as pl
from jax.experimental.pallas import tpu as pltpu
from jax.experimental.pallas import tpu_sc as plsc

_N = 10000
_NT = 10240
_E = 160000
_EP = 163840
_NCHUNK = 1280
_CPT = 80
_D = 512
_F = 128
_BR = 1000
_EPS = 1e-8

_SC_MESH = plsc.VectorSubcoreMesh(
    core_axis_name="c", subcore_axis_name="s", num_cores=2, num_subcores=16)


@functools.partial(
    pl.kernel,
    out_type=[jax.ShapeDtypeStruct((_NT, 16), jnp.float32),
              jax.ShapeDtypeStruct((_NT, 16), jnp.float32)],
    mesh=_SC_MESH,
    scratch_types=[
        pltpu.VMEM_SHARED((_NT, 16), jnp.float32),
        pltpu.VMEM((640, 16), jnp.float32),
        pltpu.VMEM((128, 16), jnp.float32),
        pltpu.VMEM((_CPT, 128), jnp.int32),
    ],
)
def _sc_degrees(sd_ref, dd_ref, dsrc_ref, ddst_ref, acc, zb, ones, idxv):
    c = lax.axis_index("c")
    s = lax.axis_index("s")

    def _zrow(i, _):
        zb[i, :] = jnp.zeros((16,), jnp.float32)
        return 0
    lax.fori_loop(0, 640, _zrow, 0)

    def _orow(i, _):
        ones[i, :] = jnp.full((16,), 1.0, jnp.float32)
        return 0
    lax.fori_loop(0, 128, _orow, 0)

    for half in range(2):
        idx_src = sd_ref if half == 0 else dd_ref
        out = dsrc_ref if half == 0 else ddst_ref

        @pl.when(c == half)
        def _():
            pltpu.sync_copy(zb, acc.at[pl.ds(s * 640, 640)])
            pltpu.sync_copy(idx_src.at[pl.ds(s * _CPT, _CPT)], idxv)

        plsc.subcore_barrier()

        @pl.when(c == half)
        def _():
            def _scat(j, _):
                pltpu.sync_copy(ones, acc.at[idxv.at[j]], add=True)
                return 0
            lax.fori_loop(0, _CPT, _scat, 0)

        plsc.subcore_barrier()

        @pl.when(c == half)
        def _():
            pltpu.sync_copy(acc.at[pl.ds(s * 640, 640)],
                            out.at[pl.ds(s * 640, 640)])


@functools.partial(
    pl.kernel,
    out_type=[jax.ShapeDtypeStruct((_NT, _F), jnp.float32)] * 4,
    mesh=_SC_MESH,
    scratch_types=[
        pltpu.VMEM_SHARED((_NT, _F), jnp.float32),
        pltpu.VMEM((_CPT * 128,), jnp.int32),
        pltpu.VMEM((4, 128), jnp.int32),
        pltpu.VMEM((2, 128, _F), jnp.float32),
        pltpu.SemaphoreType.DMA,
        pltpu.SemaphoreType.DMA,
    ],
)
def _sc_propagate(t0, t1, t2, t3, sp_ref, dp_ref,
                  o0, o1, o2, o3, acc, sidx, dring, rows, gsem, dsem):
    c = lax.axis_index("c")
    s = lax.axis_index("s")

    pltpu.sync_copy(sp_ref.at[pl.ds(s * (_CPT * 128), _CPT * 128)], sidx)

    tables = (t0, t1, t2, t3)
    outs = (o0, o1, o2, o3)
    for half in range(2):
        for b in range(2):
            tbl = tables[half * 2 + b]
            outb = outs[half * 2 + b]

            @pl.when(c == half)
            def _():
                def _zrow(i, _):
                    for k in range(8):
                        rows[0, i, pl.ds(k * 16, 16)] = (
                            jnp.zeros((16,), jnp.float32))
                    return 0
                lax.fori_loop(0, 128, _zrow, 0)
                for t in range(5):
                    pltpu.sync_copy(
                        rows.at[0], acc.at[pl.ds(s * 640 + t * 128, 128)])

            plsc.subcore_barrier()

            @pl.when(c == half)
            def _():
                for bb in range(2):
                    pltpu.async_copy(
                        dp_ref.at[pl.ds((s * _CPT + bb) * 128, 128)],
                        dring.at[bb], dsem)
                    pltpu.async_copy(
                        tbl.at[sidx.at[pl.ds(bb * 128, 128)]],
                        rows.at[bb], gsem)

                def _step(jj, _):
                    for bb in range(2):
                        cc = 2 * jj + bb
                        slot = lax.rem(cc, 4)
                        pltpu.make_async_copy(
                            dp_ref.at[pl.ds((s * _CPT + cc) * 128, 128)],
                            dring.at[slot], dsem).wait()
                        pltpu.make_async_copy(
                            tbl.at[sidx.at[pl.ds(cc * 128, 128)]],
                            rows.at[bb], gsem).wait()
                        pltpu.sync_copy(rows.at[bb], acc.at[dring.at[slot]],
                                        add=True)

                        @pl.when(cc < _CPT - 2)
                        def _():
                            slot2 = lax.rem(cc + 2, 4)
                            pltpu.async_copy(
                                dp_ref.at[pl.ds((s * _CPT + cc + 2) * 128,
                                                128)],
                                dring.at[slot2], dsem)
                            pltpu.async_copy(
                                tbl.at[sidx.at[pl.ds((cc + 2) * 128, 128)]],
                                rows.at[bb], gsem)
                    return 0
                lax.fori_loop(0, _CPT // 2, _step, 0)

            plsc.subcore_barrier()

            @pl.when(c == half)
            def _():
                pltpu.sync_copy(acc.at[pl.ds(s * 640, 640)],
                                outb.at[pl.ds(s * 640, 640)])

            plsc.subcore_barrier()


def _row(i):
    return (i, 0)


def _const(i):
    return (0, 0)


def _prep_body(x_ref, m_ref, tok_ref, ds_ref, dd_ref,
               t_refs0, t_refs1, t_refs2, t_refs3, ns_ref, nd_ref, s2_ref):
    ns = lax.rsqrt(jnp.maximum(ds_ref[:, 0:1], 1.0))
    nd = lax.rsqrt(jnp.maximum(dd_ref[:, 0:1], 1.0))
    m = m_ref[...]
    xm = jnp.where(m > 0.5, tok_ref[...], x_ref[...])
    t0 = xm * ns
    for k, ref in enumerate((t_refs0, t_refs1, t_refs2, t_refs3)):
        ref[...] = t0[:, k * _F:(k + 1) * _F]
    ns_ref[...] = ns
    nd_ref[...] = nd
    s2_ref[...] = jnp.where(m > 0.5, 0.0, ns)


def _prep(x, mf, tok, dsrc, ddst):
    return pl.pallas_call(
        _prep_body,
        grid=(_N // _BR,),
        in_specs=[
            pl.BlockSpec((_BR, _D), _row),
            pl.BlockSpec((_BR, 1), _row),
            pl.BlockSpec((1, _D), _const),
            pl.BlockSpec((_BR, 16), _row),
            pl.BlockSpec((_BR, 16), _row),
        ],
        out_specs=[pl.BlockSpec((_BR, _F), _row)] * 4 + [
            pl.BlockSpec((_BR, 1), _row)] * 3,
        out_shape=[jax.ShapeDtypeStruct((_N, _F), jnp.float32)] * 4 + [
            jax.ShapeDtypeStruct((_N, 1), jnp.float32)] * 3,
    )(x, mf, tok, dsrc, ddst)


def _mm1_body(a0, a1, a2, a3, nd_ref, ns_ref, w_ref,
              o0, o1, o2, o3):
    a = jnp.concatenate([a0[...], a1[...], a2[...], a3[...]], axis=1)
    h = jnp.dot(a * nd_ref[...], w_ref[...],
                preferred_element_type=jnp.float32)
    t = jax.nn.relu(h) * ns_ref[...]
    for k, ref in enumerate((o0, o1, o2, o3)):
        ref[...] = t[:, k * _F:(k + 1) * _F]


def _mm1(aggs, nd, ns, w):
    return pl.pallas_call(
        _mm1_body,
        grid=(_N // _BR,),
        in_specs=[pl.BlockSpec((_BR, _F), _row)] * 4 + [
            pl.BlockSpec((_BR, 1), _row),
            pl.BlockSpec((_BR, 1), _row),
            pl.BlockSpec((_D, _D), _const),
        ],
        out_specs=[pl.BlockSpec((_BR, _F), _row)] * 4,
        out_shape=[jax.ShapeDtypeStruct((_N, _F), jnp.float32)] * 4,
    )(*aggs, nd, ns, w)


def _mm2_body(a0, a1, a2, a3, nd_ref, s2_ref, w1_ref, w2_ref,
              o0, o1, o2, o3):
    a = jnp.concatenate([a0[...], a1[...], a2[...], a3[...]], axis=1)
    h = jnp.dot(a * nd_ref[...], w1_ref[...],
                preferred_element_type=jnp.float32)
    h = jax.nn.relu(h)
    rep = jnp.dot(h, w2_ref[...], preferred_element_type=jnp.float32)
    t = rep * s2_ref[...]
    for k, ref in enumerate((o0, o1, o2, o3)):
        ref[...] = t[:, k * _F:(k + 1) * _F]


def _mm2(aggs, nd, s2, w1, w2):
    return pl.pallas_call(
        _mm2_body,
        grid=(_N // _BR,),
        in_specs=[pl.BlockSpec((_BR, _F), _row)] * 4 + [
            pl.BlockSpec((_BR, 1), _row),
            pl.BlockSpec((_BR, 1), _row),
            pl.BlockSpec((_D, _D), _const),
            pl.BlockSpec((_D, _D), _const),
        ],
        out_specs=[pl.BlockSpec((_BR, _F), _row)] * 4,
        out_shape=[jax.ShapeDtypeStruct((_N, _F), jnp.float32)] * 4,
    )(*aggs, nd, s2, w1, w2)


def _final_body(a0, a1, a2, a3, nd_ref, w_ref, x_ref, m_ref, o_ref, acc_ref):
    i = pl.program_id(0)

    @pl.when(i == 0)
    def _():
        acc_ref[0] = 0.0
        acc_ref[1] = 0.0

    a = jnp.concatenate([a0[...], a1[...], a2[...], a3[...]], axis=1)
    r = jnp.dot(a * nd_ref[...], w_ref[...],
                preferred_element_type=jnp.float32)
    x = x_ref[...]
    xr = jnp.sum(x * r, axis=1, keepdims=True)
    xx = jnp.sum(x * x, axis=1, keepdims=True)
    rr = jnp.sum(r * r, axis=1, keepdims=True)
    cos = xr / ((jnp.sqrt(xx) + _EPS) * (jnp.sqrt(rr) + _EPS))
    per = (1.0 - cos) ** 2
    m = m_ref[...]
    acc_ref[0] += jnp.sum(per * m)
    acc_ref[1] += jnp.sum(m)

    @pl.when(i == pl.num_programs(0) - 1)
    def _():
        o_ref[...] = jnp.reshape(acc_ref[0] / jnp.maximum(acc_ref[1], 1.0),
                                 (1, 1))


def _final(aggs, nd, w, x, mf):
    return pl.pallas_call(
        _final_body,
        grid=(_N // _BR,),
        in_specs=[pl.BlockSpec((_BR, _F), _row)] * 4 + [
            pl.BlockSpec((_BR, 1), _row),
            pl.BlockSpec((_D, _D), _const),
            pl.BlockSpec((_BR, _D), _row),
            pl.BlockSpec((_BR, 1), _row),
        ],
        out_specs=pl.BlockSpec((1, 1), _const),
        out_shape=jax.ShapeDtypeStruct((1, 1), jnp.float32),
        scratch_shapes=[pltpu.SMEM((2,), jnp.float32)],
    )(*aggs, nd, w, x, mf)


def kernel(x, edge_index, mask, enc_mask_token, W_enc0, W_enc1, W_e2d, W_dec):
    src = edge_index[0]
    dst = edge_index[1]
    mf = mask.astype(jnp.float32)[:, None]

    npad = _EP - _E
    ar = jnp.arange(npad, dtype=jnp.int32)
    pad_pad = _N + (ar % 16)
    pad_real = (_N - 128) + (ar % 128)
    sp1 = jnp.concatenate([src, pad_real])
    dp1 = jnp.concatenate([dst, pad_pad])
    sd2d = jnp.concatenate([src, pad_pad]).reshape(_NCHUNK, 128)
    dd2d = dp1.reshape(_NCHUNK, 128)

    dsrc, ddst = _sc_degrees(sd2d, dd2d)
    t0a, t0b, t0c, t0d, ns, nd, s2 = _prep(x, mf, enc_mask_token, dsrc, ddst)
    a1 = _sc_propagate(t0a, t0b, t0c, t0d, sp1, dp1)
    t1 = _mm1(a1, nd, ns, W_enc0)
    a2 = _sc_propagate(*t1, sp1, dp1)
    t2 = _mm2(a2, nd, s2, W_enc1, W_e2d)
    a3 = _sc_propagate(*t2, sp1, dp1)
    loss = _final(a3, nd, W_dec, x, mf)
    return loss.reshape(())
